# baseline (speedup 1.0000x reference)
import functools

import jax
import jax.numpy as jnp
from jax import lax
from jax.experimental import pallas as pl
from jax.experimental.pallas import tpu as pltpu
from jax.experimental.pallas import tpu_sc as plsc

NUM_CORES = 2
NUM_SUBCORES = 16
LANES = 16
NUM_WORKERS = NUM_CORES * NUM_SUBCORES

BATCH = 16384
DIM = 64
NUM_ENT = 1000000
NUM_REL = 1000
B_PER_W = BATCH // NUM_WORKERS
CHUNK = 128
NUM_CHUNKS = B_PER_W // CHUNK
GROUPS = CHUNK // LANES
RSTRIDE = NUM_REL + 1
TBLK = 512
TSTEPS = (NUM_ENT // 2 + TBLK - 1) // TBLK
SPLIT = TSTEPS * TBLK


def _t_tc_body(xlo_ref, xhi_ref, o_ref):
    eye = jnp.float32(1.0) * (
        lax.broadcasted_iota(jnp.int32, (DIM, DIM), 0)
        == lax.broadcasted_iota(jnp.int32, (DIM, DIM), 1))
    dims = (((0,), (0,)), ((), ()))
    tr_lo = lax.dot_general(xlo_ref[...], eye, dims,
                            preferred_element_type=jnp.float32)
    tr_hi = lax.dot_general(xhi_ref[...], eye, dims,
                            preferred_element_type=jnp.float32)
    o_ref[...] = lax.concatenate([tr_lo, tr_hi], 1)


def _g_body(h_hbm, r_hbm, t_hbm, ent2_hbm, relT_hbm, out_hbm,
            hvi, rvi, tvi, hp, tp, rel2s, rel1d, hbuf, tbuf, outv, sem):
    wid = lax.axis_index("s") * NUM_CORES + lax.axis_index("c")
    base = wid * B_PER_W
    iota = lax.iota(jnp.int32, LANES)

    pltpu.sync_copy(h_hbm.at[pl.ds(base, B_PER_W)], hvi)
    pltpu.sync_copy(r_hbm.at[pl.ds(base, B_PER_W)], rvi)
    pltpu.sync_copy(t_hbm.at[pl.ds(base, B_PER_W)], tvi)

    for k in range(DIM // 8):
        pltpu.sync_copy(relT_hbm.at[pl.ds(k * 8, 8)], rel2s)
        for dd in range(8):
            d = k * 8 + dd
            for q in range(0, NUM_REL - LANES + 1, LANES):
                rel1d[pl.ds(d * RSTRIDE + q, LANES)] = rel2s[dd, pl.ds(q, LANES)]
            q = NUM_REL - LANES
            rel1d[pl.ds(d * RSTRIDE + q, LANES)] = rel2s[dd, pl.ds(q, LANES)]

    def chunk_body(c, _):
        for s in range(GROUPS):
            sl = pl.ds(c * CHUNK + s * LANES, LANES)
            hw = hvi[sl]
            tw = tvi[sl]
            hp[pl.ds(s * LANES, LANES)] = hw - jnp.where(
                hw >= SPLIT, jnp.int32(SPLIT), jnp.int32(0))
            tp[pl.ds(s * LANES, LANES)] = tw - jnp.where(
                tw >= SPLIT, jnp.int32(SPLIT), jnp.int32(0))
        cp_h = pltpu.async_copy(ent2_hbm.at[hp], hbuf, sem)
        cp_t = pltpu.async_copy(ent2_hbm.at[tp], tbuf, sem)
        cp_h.wait()
        cp_t.wait()

        def group_body(g, _):
            goff = c * CHUNK + g * LANES
            hv16 = hvi[pl.ds(goff, LANES)]
            rv16 = rvi[pl.ds(goff, LANES)]
            tv16 = tvi[pl.ds(goff, LANES)]
            acc = jnp.zeros((LANES,), jnp.float32)
            for l in range(LANES):
                row = g * LANES + l
                h_e = hv16[l]
                t_e = tv16[l]
                r_e = rv16[l]
                h_lo = (h_e >= SPLIT).astype(jnp.int32) << 6
                t_lo = (t_e >= SPLIT).astype(jnp.int32) << 6
                p = jnp.zeros((LANES,), jnp.float32)
                for m in range(DIM // LANES):
                    hv = hbuf[row, pl.ds(h_lo + m * LANES, LANES)]
                    tv = tbuf[row, pl.ds(t_lo + m * LANES, LANES)]
                    ridx = (iota + m * LANES) * RSTRIDE + r_e
                    rv = plsc.load_gather(rel1d, [ridx])
                    p = p + hv * rv * tv
                s = lax.reduce_sum(p, axes=(0,))
                acc = jnp.where(iota == l, s, acc)
            outv[pl.ds(goff, LANES)] = acc
            return 0

        lax.fori_loop(0, GROUPS, group_body, 0)
        return 0

    lax.fori_loop(0, NUM_CHUNKS, chunk_body, 0)
    pltpu.sync_copy(outv, out_hbm.at[pl.ds(base, B_PER_W)])


@jax.jit
def kernel(h, r, t, entity, relation):
    entT = entity.T
    relT = relation.T
    mesh = plsc.VectorSubcoreMesh(core_axis_name="c", subcore_axis_name="s")
    cp = pltpu.CompilerParams(needs_layout_passes=False)

    ent2 = pl.pallas_call(
        _t_tc_body,
        grid=(TSTEPS,),
        in_specs=[
            pl.BlockSpec((DIM, TBLK), lambda i: (0, i)),
            pl.BlockSpec((DIM, TBLK), lambda i: (0, i + TSTEPS)),
        ],
        out_specs=pl.BlockSpec((TBLK, 2 * DIM), lambda i: (i, 0)),
        out_shape=jax.ShapeDtypeStruct((SPLIT, 2 * DIM), jnp.float32),
    )(entT, entT)

    g_fn = functools.partial(
        pl.kernel,
        mesh=mesh,
        compiler_params=cp,
        out_type=jax.ShapeDtypeStruct((BATCH,), jnp.float32),
        scratch_types=[
            pltpu.VMEM((B_PER_W,), jnp.int32),
            pltpu.VMEM((B_PER_W,), jnp.int32),
            pltpu.VMEM((B_PER_W,), jnp.int32),
            pltpu.VMEM((CHUNK,), jnp.int32),
            pltpu.VMEM((CHUNK,), jnp.int32),
            pltpu.VMEM((8, NUM_REL), jnp.float32),
            pltpu.VMEM((DIM * RSTRIDE,), jnp.float32),
            pltpu.VMEM((CHUNK, 2 * DIM), jnp.float32),
            pltpu.VMEM((CHUNK, 2 * DIM), jnp.float32),
            pltpu.VMEM((B_PER_W,), jnp.float32),
            pltpu.SemaphoreType.DMA,
        ],
    )(_g_body)
    return g_fn(h, r, t, ent2, relT)


# speedup vs baseline: 1.8607x; 1.8607x over previous
import functools

import jax
import jax.numpy as jnp
from jax import lax
from jax.experimental import pallas as pl
from jax.experimental.pallas import tpu as pltpu
from jax.experimental.pallas import tpu_sc as plsc

NUM_CORES = 2
NUM_SUBCORES = 16
LANES = 16
NUM_WORKERS = NUM_CORES * NUM_SUBCORES

BATCH = 16384
DIM = 64
NUM_ENT = 1000000
NUM_REL = 1000
NUM_QUADS = NUM_ENT // 4
FULL_COLS = NUM_ENT // 128
TAIL_START = FULL_COLS * 128
TAIL_ENT = NUM_ENT - TAIL_START
B_PER_W = BATCH // NUM_WORKERS
CHUNK = 128
NUM_CHUNKS = B_PER_W // CHUNK
GROUPS = CHUNK // LANES
WORDS = DIM // 2
RSTRIDE = NUM_REL + 1
BASE_BLOCKS = FULL_COLS // NUM_WORKERS
EXTRA_W = FULL_COLS - BASE_BLOCKS * NUM_WORKERS
ILV = plsc.PackFormat.INTERLEAVED


def _transpose_block(inb, obuf, iota):
    def sub_body(n, _):
        col0 = n * LANES
        for d in range(LANES):
            diag = (iota + d) & 15
            e_vec = col0 + diag
            q_vec = e_vec >> 2
            cshift = (e_vec & 3) << 5
            for mp in range(2):
                rowa = 32 * mp + 2 * iota
                va = plsc.load_gather(inb, [rowa, e_vec])
                vb = plsc.load_gather(inb, [rowa + 1, e_vec])
                w = plsc.bitcast(plsc.pack(va, vb, format=ILV), jnp.int32)
                colv = cshift + (16 * mp) + iota
                plsc.store_scatter(obuf, [q_vec, colv], w)
        return 0

    lax.fori_loop(0, 8, sub_body, 0)


def _t_body(entT_hbm, ent2_hbm, inb0, inb1, ob0, ob1, sem_in, sem_out):
    wid = lax.axis_index("s") * NUM_CORES + lax.axis_index("c")
    iota = lax.iota(jnp.int32, LANES)
    inbs = (inb0, inb1)
    obs = (ob0, ob1)
    nb = BASE_BLOCKS + jnp.where(wid < EXTRA_W, 1, 0)

    def issue_in(c, buf):
        col0 = pl.multiple_of(c * 128, 128)
        pltpu.async_copy(entT_hbm.at[:, pl.ds(col0, 128)], buf, sem_in)

    def drain_in(buf):
        pltpu.make_async_copy(entT_hbm.at[pl.ds(0, DIM), pl.ds(0, 128)],
                              buf, sem_in).wait()

    def issue_out(c, buf):
        q0 = pl.multiple_of(c * 32, 32)
        pltpu.async_copy(buf, ent2_hbm.at[pl.ds(q0, 32)], sem_out)

    def drain_out(buf):
        pltpu.make_async_copy(ent2_hbm.at[pl.ds(0, 32)], buf, sem_out).wait()

    issue_in(wid, inbs[0])

    def step(i2, _):
        for b in range(2):
            i = i2 * 2 + b

            @pl.when(i < nb)
            def _(i=i, b=b):
                c = wid + NUM_WORKERS * i

                @pl.when(i + 1 < nb)
                def _():
                    issue_in(c + NUM_WORKERS, inbs[(b + 1) % 2])

                drain_in(inbs[b])

                @pl.when(i >= 2)
                def _():
                    drain_out(obs[b])

                _transpose_block(inbs[b], obs[b], iota)
                issue_out(c, obs[b])
        return 0

    lax.fori_loop(0, (BASE_BLOCKS + 2) // 2, step, 0)
    drain_out(ob0)
    drain_out(ob1)


def _g_body(h_hbm, r_hbm, t_hbm, ent2_hbm, relT_hbm, etail_hbm, out_hbm,
            hvi, rvi, tvi, hp, tp, rel2s, relp, etvm, etp, hbuf, tbuf, outv,
            sem):
    wid = lax.axis_index("s") * NUM_CORES + lax.axis_index("c")
    base = wid * B_PER_W
    iota = lax.iota(jnp.int32, LANES)

    pltpu.sync_copy(h_hbm.at[pl.ds(base, B_PER_W)], hvi)
    pltpu.sync_copy(r_hbm.at[pl.ds(base, B_PER_W)], rvi)
    pltpu.sync_copy(t_hbm.at[pl.ds(base, B_PER_W)], tvi)
    pltpu.sync_copy(etail_hbm, etvm)

    for k in range(DIM // 8):
        pltpu.sync_copy(relT_hbm.at[pl.ds(k * 8, 8)], rel2s)
        for ddp in range(4):
            j = k * 4 + ddp
            for q in list(range(0, NUM_REL - LANES + 1, LANES)) + [NUM_REL - LANES]:
                a = rel2s[2 * ddp, pl.ds(q, LANES)]
                b = rel2s[2 * ddp + 1, pl.ds(q, LANES)]
                w = plsc.bitcast(plsc.pack(a, b, format=ILV), jnp.int32)
                relp[pl.ds(j * RSTRIDE + q, LANES)] = w

    def tail_pack(e, _):
        for mp in range(2):
            rowa = 32 * mp + 2 * iota
            ai = plsc.load_gather(etvm, [jnp.full((LANES,), 0, jnp.int32) + e, rowa])
            bi = plsc.load_gather(etvm, [jnp.full((LANES,), 0, jnp.int32) + e, rowa + 1])
            a = plsc.bitcast(ai, jnp.float32)
            b = plsc.bitcast(bi, jnp.float32)
            w = plsc.bitcast(plsc.pack(a, b, format=ILV), jnp.int32)
            etp[e, pl.ds(mp * LANES, LANES)] = w
        return 0

    lax.fori_loop(0, TAIL_ENT, tail_pack, 0)

    def chunk_body(c, _):
        for s in range(GROUPS):
            sl = pl.ds(c * CHUNK + s * LANES, LANES)
            hp[pl.ds(s * LANES, LANES)] = hvi[sl] >> 2
            tp[pl.ds(s * LANES, LANES)] = tvi[sl] >> 2
        cp_h = pltpu.async_copy(ent2_hbm.at[hp], hbuf, sem)
        cp_t = pltpu.async_copy(ent2_hbm.at[tp], tbuf, sem)
        cp_h.wait()
        cp_t.wait()

        def group_body(g, _):
            goff = c * CHUNK + g * LANES
            hv16 = hvi[pl.ds(goff, LANES)]
            rv16 = rvi[pl.ds(goff, LANES)]
            tv16 = tvi[pl.ds(goff, LANES)]
            acc = jnp.zeros((LANES,), jnp.float32)
            for l in range(LANES):
                row = g * LANES + l
                h_e = hv16[l]
                t_e = tv16[l]
                r_e = rv16[l]
                h_lo = (h_e & 3) << 5
                t_lo = (t_e & 3) << 5
                h_tail = h_e >= TAIL_START
                t_tail = t_e >= TAIL_START
                h_te = lax.max(h_e - TAIL_START, 0)
                t_te = lax.max(t_e - TAIL_START, 0)
                p = jnp.zeros((LANES,), jnp.float32)
                for mp in range(2):
                    wh = jnp.where(h_tail, etp[h_te, pl.ds(mp * LANES, LANES)],
                                   hbuf[row, pl.ds(h_lo + mp * LANES, LANES)])
                    wt = jnp.where(t_tail, etp[t_te, pl.ds(mp * LANES, LANES)],
                                   tbuf[row, pl.ds(t_lo + mp * LANES, LANES)])
                    ridx = (iota + mp * LANES) * RSTRIDE + r_e
                    wr = plsc.load_gather(relp, [ridx])
                    ha, hb = plsc.unpack(plsc.bitcast(wh, jnp.bfloat16), format=ILV)
                    ta, tb = plsc.unpack(plsc.bitcast(wt, jnp.bfloat16), format=ILV)
                    ra, rb = plsc.unpack(plsc.bitcast(wr, jnp.bfloat16), format=ILV)
                    p = p + ha * ra * ta + hb * rb * tb
                s = lax.reduce_sum(p, axes=(0,))
                acc = jnp.where(iota == l, s, acc)
            outv[pl.ds(goff, LANES)] = acc
            return 0

        lax.fori_loop(0, GROUPS, group_body, 0)
        return 0

    lax.fori_loop(0, NUM_CHUNKS, chunk_body, 0)
    pltpu.sync_copy(outv, out_hbm.at[pl.ds(base, B_PER_W)])


@jax.jit
def kernel(h, r, t, entity, relation):
    entT = entity.T
    relT = relation.T
    etail = lax.bitcast_convert_type(entity[TAIL_START:], jnp.int32)
    mesh = plsc.VectorSubcoreMesh(core_axis_name="c", subcore_axis_name="s")
    cp = pltpu.CompilerParams(needs_layout_passes=False)

    t_fn = functools.partial(
        pl.kernel,
        mesh=mesh,
        compiler_params=cp,
        out_type=jax.ShapeDtypeStruct((NUM_QUADS, 2 * DIM), jnp.int32),
        scratch_types=[
            pltpu.VMEM((DIM, 128), jnp.float32),
            pltpu.VMEM((DIM, 128), jnp.float32),
            pltpu.VMEM((32, 2 * DIM), jnp.int32),
            pltpu.VMEM((32, 2 * DIM), jnp.int32),
            pltpu.SemaphoreType.DMA,
            pltpu.SemaphoreType.DMA,
        ],
    )(_t_body)
    ent2 = t_fn(entT)

    g_fn = functools.partial(
        pl.kernel,
        mesh=mesh,
        compiler_params=cp,
        out_type=jax.ShapeDtypeStruct((BATCH,), jnp.float32),
        scratch_types=[
            pltpu.VMEM((B_PER_W,), jnp.int32),
            pltpu.VMEM((B_PER_W,), jnp.int32),
            pltpu.VMEM((B_PER_W,), jnp.int32),
            pltpu.VMEM((CHUNK,), jnp.int32),
            pltpu.VMEM((CHUNK,), jnp.int32),
            pltpu.VMEM((8, NUM_REL), jnp.float32),
            pltpu.VMEM((WORDS * RSTRIDE,), jnp.int32),
            pltpu.VMEM((TAIL_ENT, DIM), jnp.int32),
            pltpu.VMEM((TAIL_ENT, WORDS), jnp.int32),
            pltpu.VMEM((CHUNK, 2 * DIM), jnp.int32),
            pltpu.VMEM((CHUNK, 2 * DIM), jnp.int32),
            pltpu.VMEM((B_PER_W,), jnp.float32),
            pltpu.SemaphoreType.DMA,
        ],
    )(_g_body)
    return g_fn(h, r, t, ent2, relT, etail)
